# baseline (speedup 1.0000x reference)
import functools

import jax
import jax.numpy as jnp
from jax import lax
from jax.experimental import pallas as pl
from jax.experimental.pallas import tpu as pltpu
from jax.experimental.pallas import tpu_sc as plsc

_NC = 2
_NS = 16
_NW = _NC * _NS
_L = 128
_D = 128
_CH = 256
_NBUF = 3
_BLK_B = 128
_B_TC = 2560


def _sc_add_tail(x_hbm, pos_hbm, out_hbm, buf, pos_v, sem_in, sem_out):
    tail_rows = out_hbm.shape[0]
    head_rows = x_hbm.shape[0] - tail_rows
    rows_per_w = tail_rows // _NW
    n_chunks = rows_per_w // _CH

    cid = lax.axis_index("c")
    sid = lax.axis_index("s")
    wid = sid * _NC + cid
    w_base = head_rows + wid * rows_per_w
    o_base = wid * rows_per_w

    pltpu.sync_copy(pos_hbm.at[pl.ds(0, _L)], pos_v)

    def start_in(c, b):
        pltpu.async_copy(
            x_hbm.at[pl.ds(w_base + c * _CH, _CH)], buf.at[b], sem_in.at[b])

    def wait_in(c, b):
        pltpu.make_async_copy(
            x_hbm.at[pl.ds(w_base + c * _CH, _CH)], buf.at[b],
            sem_in.at[b]).wait()

    def start_out(c, b):
        pltpu.async_copy(
            buf.at[b], out_hbm.at[pl.ds(o_base + c * _CH, _CH)],
            sem_out.at[b])

    def wait_out(c, b):
        pltpu.make_async_copy(
            buf.at[b], out_hbm.at[pl.ds(o_base + c * _CH, _CH)],
            sem_out.at[b]).wait()

    start_in(0, 0)
    start_in(1, 1)

    def body(c, _):
        b = lax.rem(c, _NBUF)
        wait_in(c, b)

        @plsc.parallel_loop(0, _CH)
        def _(r):
            for j in range(_D // 16):
                sl = pl.ds(j * 16, 16)
                plsc.addupdate(buf.at[b, r, sl], pos_v[lax.rem(r, _L), sl])

        bp = lax.rem(c + 2, _NBUF)

        @pl.when(c >= 1)
        def _():
            wait_out(c - 1, bp)

        @pl.when(c + 2 < n_chunks)
        def _():
            start_in(c + 2, bp)

        start_out(c, b)
        return 0

    lax.fori_loop(0, n_chunks, body, 0)
    wait_out(n_chunks - 1, lax.rem(n_chunks - 1, _NBUF))


def _tc_add(x_ref, pos_ref, o_ref):
    o_ref[...] = x_ref[...] + pos_ref[0:_L, :][None, :, :]


def kernel(x, pos_emb):
    B, L, D = x.shape
    x2 = x.reshape(B * L, D)
    tail_rows = (B - _B_TC) * L

    sc_run = functools.partial(
        pl.kernel,
        out_type=jax.ShapeDtypeStruct((tail_rows, D), x.dtype),
        mesh=plsc.VectorSubcoreMesh(core_axis_name="c", subcore_axis_name="s"),
        scratch_types=[
            pltpu.VMEM((_NBUF, _CH, D), jnp.float32),
            pltpu.VMEM((_L, D), jnp.float32),
            pltpu.SemaphoreType.DMA((_NBUF,)),
            pltpu.SemaphoreType.DMA((_NBUF,)),
        ],
    )(_sc_add_tail)
    sc_out = sc_run(x2, pos_emb)

    tc_out = pl.pallas_call(
        _tc_add,
        grid=(_B_TC // _BLK_B,),
        in_specs=[
            pl.BlockSpec((_BLK_B, L, D), lambda i: (i, 0, 0)),
            pl.BlockSpec(pos_emb.shape, lambda i: (0, 0)),
        ],
        out_specs=pl.BlockSpec((_BLK_B, L, D), lambda i: (i, 0, 0)),
        out_shape=jax.ShapeDtypeStruct((_B_TC, L, D), x.dtype),
    )(x, pos_emb)

    return jnp.concatenate(
        [tc_out.reshape(_B_TC * L, D), sc_out], axis=0).reshape(B, L, D)


# speedup vs baseline: 1.9237x; 1.9237x over previous
import functools

import jax
import jax.numpy as jnp
from jax import lax
from jax.experimental import pallas as pl
from jax.experimental.pallas import tpu as pltpu
from jax.experimental.pallas import tpu_sc as plsc

_BLK_B = 128


def _sc_gather(pos_hbm, out_hbm, idx_v, row_v, sem):
    cid = lax.axis_index("c")
    sid = lax.axis_index("s")
    L = out_hbm.shape[0]

    @pl.when(jnp.logical_and(cid == 0, sid == 0))
    def _():
        for i in range(L // 16):
            idx_v[pl.ds(i * 16, 16)] = lax.iota(jnp.int32, 16) + i * 16
        pltpu.async_copy(pos_hbm.at[idx_v], row_v, sem).wait()
        pltpu.sync_copy(row_v, out_hbm)


def _tc_add(x_ref, pos_ref, o_ref):
    o_ref[...] = x_ref[...] + pos_ref[...][None, :, :]


def kernel(x, pos_emb):
    B, L, D = x.shape
    pos = pl.kernel(
        _sc_gather,
        out_type=jax.ShapeDtypeStruct((L, D), pos_emb.dtype),
        mesh=plsc.VectorSubcoreMesh(
            core_axis_name="c", subcore_axis_name="s", num_cores=1),
        scratch_types=[
            pltpu.VMEM((L,), jnp.int32),
            pltpu.VMEM((L, D), pos_emb.dtype),
            pltpu.SemaphoreType.DMA,
        ],
    )(pos_emb)
    return pl.pallas_call(
        _tc_add,
        grid=(B // _BLK_B,),
        in_specs=[
            pl.BlockSpec((_BLK_B, L, D), lambda i: (i, 0, 0)),
            pl.BlockSpec((L, D), lambda i: (0, 0)),
        ],
        out_specs=pl.BlockSpec((_BLK_B, L, D), lambda i: (i, 0, 0)),
        out_shape=jax.ShapeDtypeStruct((B, L, D), x.dtype),
    )(x, pos)
